# baseline (speedup 1.0000x reference)
import functools

import numpy as np

import jax
import jax.numpy as jnp
from jax import lax
from jax.experimental import pallas as pl
from jax.experimental.pallas import tpu as pltpu

_N_BINS = 15


def _ece_kernel(logits_ref, labels_ref, out_ref, acc_ref, *, n_total, n_blocks):
    i = pl.program_id(0)

    @pl.when(i == 0)
    def _init():
        acc_ref[...] = jnp.zeros_like(acc_ref)

    x = logits_ref[...]
    lab = labels_ref[...]
    b, c = x.shape

    m = jnp.max(x, axis=1, keepdims=True)
    s = jnp.sum(jnp.exp(x - m), axis=1, keepdims=True)
    conf = 1.0 / s
    idx = lax.broadcasted_iota(jnp.int32, (b, c), 1)
    first_max = jnp.min(jnp.where(x == m, idx, c), axis=1, keepdims=True)
    acc = (first_max == lab).astype(jnp.float32)

    ii = lax.broadcasted_iota(jnp.int32, (1, _N_BINS), 1).astype(jnp.float32)
    lo = ii / float(_N_BINS)
    hi = (ii + 1.0) / float(_N_BINS)
    mask = ((conf > lo) & (conf <= hi)).astype(jnp.float32)
    acc_ref[0, :] += jnp.sum(mask, axis=0)
    acc_ref[1, :] += jnp.sum(conf * mask, axis=0)
    acc_ref[2, :] += jnp.sum(acc * mask, axis=0)

    @pl.when(i == n_blocks - 1)
    def _finish():
        cnt = acc_ref[0, :]
        safe = jnp.maximum(cnt, 1.0)
        avg_conf = acc_ref[1, :] / safe
        avg_acc = acc_ref[2, :] / safe
        prop = cnt / n_total
        contrib = jnp.abs(avg_conf - avg_acc) * prop
        out_ref[...] = jnp.sum(jnp.where(prop > 0, contrib, 0.0)).reshape(1, 1)


def kernel(logits, labels):
    n, c = logits.shape
    labels2 = labels.astype(jnp.int32).reshape(n, 1)
    blk = 4000
    n_blocks = n // blk
    out = pl.pallas_call(
        functools.partial(_ece_kernel, n_total=float(n), n_blocks=n_blocks),
        grid=(n_blocks,),
        in_specs=[
            pl.BlockSpec((blk, c), lambda i: (i, 0)),
            pl.BlockSpec((blk, 1), lambda i: (i, 0)),
        ],
        out_specs=pl.BlockSpec((1, 1), lambda i: (0, 0)),
        out_shape=jax.ShapeDtypeStruct((1, 1), jnp.float32),
        scratch_shapes=[pltpu.VMEM((3, _N_BINS), jnp.float32)],
        compiler_params=pltpu.CompilerParams(
            dimension_semantics=("arbitrary",)),
    )(logits, labels2)
    return out.reshape(1)
